# baseline (speedup 1.0000x reference)
import functools

import jax
import jax.numpy as jnp
from jax import lax
from jax.experimental import pallas as pl
from jax.experimental.pallas import tpu as pltpu
from jax.experimental.pallas import tpu_sc as plsc

VOCAB = 100000
EMBED_DIM = 32
BATCH = 4096
SEQ = 200

NC, NS = 2, 16
NW = NC * NS
LANES = 16
GS = 8
NG = SEQ // GS
NSLOT = 3

_MESH = plsc.VectorSubcoreMesh(
    core_axis_name="c", subcore_axis_name="s", num_cores=NC, num_subcores=NS
)


@functools.partial(
    pl.kernel,
    out_type=jax.ShapeDtypeStruct((SEQ, EMBED_DIM, BATCH), jnp.float32),
    mesh=_MESH,
    compiler_params=pltpu.CompilerParams(use_tc_tiling_on_sc=False,
                                         needs_layout_passes=False),
    scratch_types=[
        pltpu.VMEM_SHARED((NSLOT, GS, BATCH), jnp.int32),
        pltpu.VMEM((VOCAB,), jnp.float32),
        pltpu.VMEM((BATCH,), jnp.int32),
        pltpu.VMEM((BATCH,), jnp.int32),
        pltpu.VMEM((2, BATCH), jnp.float32),
        pltpu.VMEM((2, BATCH), jnp.float32),
        pltpu.SemaphoreType.DMA,
        pltpu.SemaphoreType.DMA,
        pltpu.SemaphoreType.DMA,
        pltpu.SemaphoreType.DMA,
        pltpu.SemaphoreType.DMA,
    ],
)
def _lookup_kernel(idx_hbm, table_t_hbm, out_hbm, sidx, tcol,
                   i0, i1, o0, o1,
                   hs, ls0, ls1, os0, os1):
    sid = lax.axis_index("s")
    cid = lax.axis_index("c")
    e = sid * NC + cid
    il = (i0, i1)
    outb = (o0, o1)
    lsem = (ls0, ls1)
    osem = (os0, os1)

    def idx_rows(g):
        return idx_hbm.at[pl.ds(g * GS, GS)]

    @pl.when(sid == 0)
    def _():
        pltpu.async_copy(idx_rows(0), sidx.at[0], hs)

    pltpu.sync_copy(table_t_hbm.at[e], tcol)

    @pl.when(sid == 0)
    def _():
        pltpu.make_async_copy(idx_rows(0), sidx.at[0], hs).wait()
        pltpu.async_copy(idx_rows(1), sidx.at[1], hs)

    plsc.subcore_barrier()

    pltpu.async_copy(sidx.at[0, 0], i0, ls0)
    pltpu.async_copy(sidx.at[0, 1], i1, ls1)

    @pl.loop(0, NG)
    def _group(g):
        @pl.when(g + 1 < NG)
        def _():
            @pl.when(sid == 0)
            def _():
                pltpu.make_async_copy(idx_rows(0), sidx.at[0], hs).wait()

        plsc.subcore_barrier()

        @pl.when(g + 2 < NG)
        def _():
            @pl.when(sid == 0)
            def _():
                pltpu.async_copy(idx_rows(g + 2),
                                 sidx.at[lax.rem(g + 2, NSLOT)], hs)

        @pl.loop(0, GS, step=4)
        def _srow(ro):
            for b in range(4):
                s = g * GS + ro + b
                b2 = b % 2
                op = b // 2
                r = b % 2

                if r == 0:
                    @pl.when(s >= 4)
                    def _():
                        pltpu.make_async_copy(
                            outb[op], out_hbm.at[pl.ds(0, 2), 0],
                            osem[op]).wait()

                pltpu.make_async_copy(sidx.at[0, 0], il[b2], lsem[b2]).wait()

                U = 8
                @pl.loop(0, BATCH // (LANES * U))
                def _grp(j):
                    base = j * (LANES * U)
                    ivs = [il[b2][pl.ds(base + k * LANES, LANES)]
                           for k in range(U)]
                    rs = [plsc.load_gather(tcol, [iv]) for iv in ivs]
                    for k in range(U):
                        outb[op][r, pl.ds(base + k * LANES, LANES)] = rs[k]

                @pl.when(s + 2 < SEQ)
                def _():
                    pltpu.async_copy(
                        sidx.at[lax.rem(lax.div(s + 2, GS), NSLOT),
                                lax.rem(s + 2, GS)],
                        il[b2], lsem[b2])

                if r == 1:
                    pltpu.async_copy(outb[op],
                                     out_hbm.at[pl.ds(s - 1, 2), e],
                                     osem[op])

    for op in range(2):
        pltpu.make_async_copy(outb[op], out_hbm.at[pl.ds(0, 2), 0],
                              osem[op]).wait()


def kernel(words_seq, table):
    idx_t = words_seq.T
    table_t = table.T
    out = _lookup_kernel(idx_t, table_t)
    return out.transpose(2, 0, 1)


# speedup vs baseline: 1.0329x; 1.0329x over previous
import functools

import jax
import jax.numpy as jnp
from jax import lax
from jax.experimental import pallas as pl
from jax.experimental.pallas import tpu as pltpu
from jax.experimental.pallas import tpu_sc as plsc

VOCAB = 100000
EMBED_DIM = 32
BATCH = 4096
SEQ = 200

NC, NS = 2, 16
NW = NC * NS
LANES = 16
GS = 8
NG = SEQ // GS
NSLOT = 3

_MESH = plsc.VectorSubcoreMesh(
    core_axis_name="c", subcore_axis_name="s", num_cores=NC, num_subcores=NS
)


@functools.partial(
    pl.kernel,
    out_type=jax.ShapeDtypeStruct((SEQ, EMBED_DIM, BATCH), jnp.float32),
    mesh=_MESH,
    compiler_params=pltpu.CompilerParams(use_tc_tiling_on_sc=False,
                                         needs_layout_passes=False),
    scratch_types=[
        pltpu.VMEM_SHARED((NSLOT, GS, BATCH), jnp.int32),
        pltpu.VMEM((VOCAB,), jnp.float32),
        pltpu.VMEM((2, BATCH), jnp.int32),
        pltpu.VMEM((2, BATCH), jnp.int32),
        pltpu.VMEM((BATCH,), jnp.float32),
        pltpu.VMEM((BATCH,), jnp.float32),
        pltpu.SemaphoreType.DMA,
        pltpu.SemaphoreType.DMA,
        pltpu.SemaphoreType.DMA,
        pltpu.SemaphoreType.DMA,
        pltpu.SemaphoreType.DMA,
    ],
)
def _lookup_kernel(idx_hbm, table_t_hbm, out_hbm, sidx, tcol,
                   i0, i1, o0, o1,
                   hs, ls0, ls1, os0, os1):
    sid = lax.axis_index("s")
    cid = lax.axis_index("c")
    e = sid * NC + cid
    il = (i0, i1)
    outb = (o0, o1)
    lsem = (ls0, ls1)
    osem = (os0, os1)

    def idx_rows(g):
        return idx_hbm.at[pl.ds(g * GS, GS)]

    @pl.when(sid == 0)
    def _():
        pltpu.async_copy(idx_rows(0), sidx.at[0], hs)

    pltpu.sync_copy(table_t_hbm.at[e], tcol)

    @pl.when(sid == 0)
    def _():
        pltpu.make_async_copy(idx_rows(0), sidx.at[0], hs).wait()
        pltpu.async_copy(idx_rows(1), sidx.at[1], hs)

    plsc.subcore_barrier()

    pltpu.async_copy(sidx.at[0, pl.ds(0, 2)], i0, ls0)
    pltpu.async_copy(sidx.at[0, pl.ds(2, 2)], i1, ls1)

    @pl.loop(0, NG)
    def _group(g):
        @pl.when(g + 1 < NG)
        def _():
            @pl.when(sid == 0)
            def _():
                pltpu.make_async_copy(idx_rows(0), sidx.at[0], hs).wait()

        plsc.subcore_barrier()

        @pl.when(g + 2 < NG)
        def _():
            @pl.when(sid == 0)
            def _():
                pltpu.async_copy(idx_rows(g + 2),
                                 sidx.at[lax.rem(g + 2, NSLOT)], hs)

        @pl.loop(0, GS, step=4)
        def _srow(ro):
            for b in range(4):
                s = g * GS + ro + b
                p = b // 2
                r = b % 2
                ob = b % 2

                @pl.when(s >= 2)
                def _():
                    pltpu.make_async_copy(outb[ob], out_hbm.at[0, 0],
                                          osem[ob]).wait()

                if r == 0:
                    pltpu.make_async_copy(sidx.at[0, pl.ds(0, 2)],
                                          il[p], lsem[p]).wait()

                U = 16
                @pl.loop(0, BATCH // (LANES * U))
                def _grp(j):
                    base = j * (LANES * U)
                    ivs = [il[p][r, pl.ds(base + k * LANES, LANES)]
                           for k in range(U)]
                    rs = [plsc.load_gather(tcol, [iv]) for iv in ivs]
                    for k in range(U):
                        outb[ob][pl.ds(base + k * LANES, LANES)] = rs[k]

                if r == 1:
                    @pl.when(s + 4 < SEQ)
                    def _():
                        pltpu.async_copy(
                            sidx.at[lax.rem(lax.div(s + 3, GS), NSLOT),
                                    pl.ds(lax.rem(s + 3, GS), 2)],
                            il[p], lsem[p])

                pltpu.async_copy(outb[ob], out_hbm.at[s, e], osem[ob])

    for b in range(2):
        pltpu.make_async_copy(outb[b], out_hbm.at[0, 0], osem[b]).wait()


def kernel(words_seq, table):
    idx_t = words_seq.T
    table_t = table.T
    out = _lookup_kernel(idx_t, table_t)
    return out.transpose(2, 0, 1)
